# Initial kernel scaffold; baseline (speedup 1.0000x reference)
import jax
import jax.numpy as jnp
from jax.experimental import pallas as pl


def kernel(inputs, epsilon):
    raise NotImplementedError("write your pallas kernel here")


# speedup vs baseline: 1066.2263x; 1066.2263x over previous
import jax
import jax.numpy as jnp
from jax.experimental import pallas as pl

_B = 1024
_L = 512
_M = 64
_TINY = 1e-38


def _qgps_body(x_ref, eps_ref, out_ref):
    eps0 = eps_ref[0]
    eps1 = eps_ref[1]

    la0 = jnp.log(jnp.maximum(jnp.abs(eps0), _TINY))
    la1 = jnp.log(jnp.maximum(jnp.abs(eps1), _TINY))
    n0 = (eps0 < 0).astype(jnp.float32)
    n1 = (eps1 < 0).astype(jnp.float32)

    dmat = jnp.concatenate([la1 - la0, n1 - n0], axis=0)
    base_log = jnp.sum(la0, axis=-1)
    base_neg = jnp.sum(n0, axis=-1)

    xf = x_ref[...].astype(jnp.float32)
    y = jax.lax.dot_general(
        xf, dmat,
        dimension_numbers=(((1,), (1,)), ((), ())),
        preferred_element_type=jnp.float32,
        precision=jax.lax.Precision.HIGHEST,
    )

    logmag = y[:, :_M] + base_log[None, :]
    parity = y[:, _M:] + base_neg[None, :]
    odd = parity - 2.0 * jnp.floor(parity * 0.5)
    sign = 1.0 - 2.0 * odd
    prod = sign * jnp.exp(logmag)
    out_ref[...] = jnp.sum(prod, axis=-1)


def kernel(inputs, epsilon):
    x = inputs.astype(jnp.int32)
    return pl.pallas_call(
        _qgps_body,
        out_shape=jax.ShapeDtypeStruct((_B,), jnp.float32),
    )(x, epsilon)
